# baseline (speedup 1.0000x reference)
import functools

import jax
import jax.numpy as jnp
from jax.experimental import pallas as pl
from jax.experimental.pallas import tpu as pltpu
from jax.sharding import Mesh, PartitionSpec as P

_G = 64


def _pool_kernel(x_ref, b_ref, w1_ref, b1_ref, w2_ref, b2_ref,
                 acc_out, m_out, d_out, m_ref, d_ref, acc_ref,
                 *, num_segments):
    i = pl.program_id(0)
    nb = pl.num_programs(0)
    g = num_segments

    @pl.when(i == 0)
    def _init():
        m_ref[...] = jnp.full(m_ref.shape, -jnp.inf, dtype=jnp.float32)
        d_ref[...] = jnp.zeros(d_ref.shape, dtype=jnp.float32)
        acc_ref[...] = jnp.zeros(acc_ref.shape, dtype=jnp.float32)

    x = x_ref[...]
    xb = x.astype(jnp.bfloat16)
    seg = b_ref[0]
    bsz = x.shape[0]

    h = jnp.tanh(
        jnp.dot(xb, w1_ref[...], preferred_element_type=jnp.float32)
        + b1_ref[...])
    s_t = jax.lax.dot_general(
        w2_ref[...], h, (((1,), (1,)), ((), ())),
        preferred_element_type=jnp.float32) + b2_ref[...]

    blk_max = jnp.max(s_t)
    m_old = m_ref[...]
    m_new = jnp.maximum(m_old, blk_max)
    scale_old = jnp.exp(m_old - m_new)
    scale_blk = jnp.exp(blk_max - m_new)

    e_t = jnp.exp(s_t - blk_max)
    seg_ids = jax.lax.broadcasted_iota(jnp.int32, (g, bsz), 0)
    p = jnp.where(seg_ids == seg, e_t, 0.0).astype(jnp.bfloat16)

    pd = jnp.dot(p, jnp.ones((bsz, 1), jnp.bfloat16),
                 preferred_element_type=jnp.float32)
    pa = jnp.dot(p, xb, preferred_element_type=jnp.float32)

    m_ref[...] = m_new
    d_ref[...] = d_ref[...] * scale_old + scale_blk * pd
    acc_ref[...] = acc_ref[...] * scale_old + scale_blk * pa

    @pl.when(i == nb - 1)
    def _finish():
        acc_out[...] = acc_ref[...]
        m_out[...] = m_ref[...]
        d_out[...] = d_ref[...]


def _partials(x, batch32, w1b, b1r, w2r, b2r, block):
    n, d = x.shape
    g = _G
    nb = n // block
    grid_spec = pltpu.PrefetchScalarGridSpec(
        num_scalar_prefetch=0,
        grid=(nb,),
        in_specs=[
            pl.BlockSpec((block, d), lambda i: (i, 0)),
            pl.BlockSpec((1, 1, block), lambda i: (i, 0, 0)),
            pl.BlockSpec((d, d), lambda i: (0, 0)),
            pl.BlockSpec((1, d), lambda i: (0, 0)),
            pl.BlockSpec((1, d), lambda i: (0, 0)),
            pl.BlockSpec((1, 1), lambda i: (0, 0)),
        ],
        out_specs=[
            pl.BlockSpec((g, d), lambda i: (0, 0)),
            pl.BlockSpec((g, 1), lambda i: (0, 0)),
            pl.BlockSpec((g, 1), lambda i: (0, 0)),
        ],
        scratch_shapes=[
            pltpu.VMEM((g, 1), jnp.float32),
            pltpu.VMEM((g, 1), jnp.float32),
            pltpu.VMEM((g, d), jnp.float32),
        ],
    )
    return pl.pallas_call(
        functools.partial(_pool_kernel, num_segments=g),
        grid_spec=grid_spec,
        out_shape=[
            jax.ShapeDtypeStruct((g, d), jnp.float32),
            jax.ShapeDtypeStruct((g, 1), jnp.float32),
            jax.ShapeDtypeStruct((g, 1), jnp.float32),
        ],
        compiler_params=pltpu.CompilerParams(
            dimension_semantics=("arbitrary",),
        ),
    )(x, batch32, w1b, b1r, w2r, b2r)


def kernel(x, batch, W1, b1, W2, b2):
    n, d = x.shape
    devs = jax.devices()
    ndev = 2 if len(devs) >= 2 and n % 2 == 0 else 1
    block = 2000 if ndev == 2 else 4000
    assert (n // ndev) % block == 0

    batch32 = batch.astype(jnp.int32).reshape(n // block, 1, block)
    w1b = W1.astype(jnp.bfloat16)
    b1r = b1.reshape(1, d)
    w2r = W2.reshape(1, d)
    b2r = b2.reshape(1, 1)

    def shard_fn(xs, bs, w1s, b1s, w2s, b2s):
        acc, m, d_ = _partials(xs, bs, w1s, b1s, w2s, b2s, block)
        m_star = jax.lax.pmax(m, "i")
        m_safe = jnp.where(jnp.isfinite(m_star), m_star, 0.0)
        r = jnp.exp(m - m_safe)
        d_tot = jax.lax.psum(d_ * r, "i")
        acc_tot = jax.lax.psum(acc * r, "i")
        return acc_tot / jnp.where(d_tot > 0, d_tot, 1.0)

    mesh = Mesh(devs[:ndev], ("i",))
    return jax.shard_map(
        shard_fn, mesh=mesh,
        in_specs=(P("i"), P("i"), P(), P(), P(), P()),
        out_specs=P(),
        check_vma=False,
    )(x, batch32, w1b, b1r, w2r, b2r)


# speedup vs baseline: 11.5729x; 11.5729x over previous
import functools

import jax
import jax.numpy as jnp
from jax.experimental import pallas as pl
from jax.experimental.pallas import tpu as pltpu

_G = 64


def _pool_kernel(x_ref, b_ref, w1_ref, b1_ref, w2_ref, b2_ref,
                 acc_out, m_out, d_out, m_ref, d_ref, acc_ref,
                 *, num_segments):
    i = pl.program_id(0)
    nb = pl.num_programs(0)
    g = num_segments

    @pl.when(i == 0)
    def _init():
        m_ref[...] = jnp.full(m_ref.shape, -jnp.inf, dtype=jnp.float32)
        d_ref[...] = jnp.zeros(d_ref.shape, dtype=jnp.float32)
        acc_ref[...] = jnp.zeros(acc_ref.shape, dtype=jnp.float32)

    x = x_ref[...]
    xb = x.astype(jnp.bfloat16)
    seg = b_ref[0]
    bsz = x.shape[0]

    h = jnp.tanh(
        jnp.dot(xb, w1_ref[...], preferred_element_type=jnp.float32)
        + b1_ref[...])
    s_t = jax.lax.dot_general(
        w2_ref[...], h, (((1,), (1,)), ((), ())),
        preferred_element_type=jnp.float32) + b2_ref[...]

    blk_max = jnp.max(s_t)
    m_old = m_ref[...]
    m_new = jnp.maximum(m_old, blk_max)
    scale_old = jnp.exp(m_old - m_new)
    scale_blk = jnp.exp(blk_max - m_new)

    e_t = jnp.exp(s_t - blk_max)
    seg_ids = jax.lax.broadcasted_iota(jnp.int32, (g, bsz), 0)
    p = jnp.where(seg_ids == seg, e_t, 0.0).astype(jnp.bfloat16)

    pd = jnp.dot(p, jnp.ones((bsz, 1), jnp.bfloat16),
                 preferred_element_type=jnp.float32)
    pa = jnp.dot(p, xb, preferred_element_type=jnp.float32)

    m_ref[...] = m_new
    d_ref[...] = d_ref[...] * scale_old + scale_blk * pd
    acc_ref[...] = acc_ref[...] * scale_old + scale_blk * pa

    @pl.when(i == nb - 1)
    def _finish():
        d_fin = d_ref[...]
        acc_out[...] = acc_ref[...] / jnp.where(d_fin > 0, d_fin, 1.0)
        m_out[...] = m_ref[...]
        d_out[...] = d_fin


def _partials(x, batch32, w1b, b1r, w2r, b2r, block):
    n, d = x.shape
    g = _G
    nb = n // block
    grid_spec = pltpu.PrefetchScalarGridSpec(
        num_scalar_prefetch=0,
        grid=(nb,),
        in_specs=[
            pl.BlockSpec((block, d), lambda i: (i, 0)),
            pl.BlockSpec((1, 1, block), lambda i: (i, 0, 0)),
            pl.BlockSpec((d, d), lambda i: (0, 0)),
            pl.BlockSpec((1, d), lambda i: (0, 0)),
            pl.BlockSpec((1, d), lambda i: (0, 0)),
            pl.BlockSpec((1, 1), lambda i: (0, 0)),
        ],
        out_specs=[
            pl.BlockSpec((g, d), lambda i: (0, 0)),
            pl.BlockSpec((g, 1), lambda i: (0, 0)),
            pl.BlockSpec((g, 1), lambda i: (0, 0)),
        ],
        scratch_shapes=[
            pltpu.VMEM((g, 1), jnp.float32),
            pltpu.VMEM((g, 1), jnp.float32),
            pltpu.VMEM((g, d), jnp.float32),
        ],
    )
    return pl.pallas_call(
        functools.partial(_pool_kernel, num_segments=g),
        grid_spec=grid_spec,
        out_shape=[
            jax.ShapeDtypeStruct((g, d), jnp.float32),
            jax.ShapeDtypeStruct((g, 1), jnp.float32),
            jax.ShapeDtypeStruct((g, 1), jnp.float32),
        ],
        compiler_params=pltpu.CompilerParams(
            dimension_semantics=("arbitrary",),
        ),
    )(x, batch32, w1b, b1r, w2r, b2r)


def kernel(x, batch, W1, b1, W2, b2):
    n, d = x.shape
    block = 4000
    assert n % block == 0

    batch32 = batch.astype(jnp.int32).reshape(n // block, 1, block)
    w1b = W1.astype(jnp.bfloat16)
    b1r = b1.reshape(1, d)
    w2r = W2.reshape(1, d)
    b2r = b2.reshape(1, 1)

    out, _, _ = _partials(x, batch32, w1b, b1r, w2r, b2r, block)
    return out


# speedup vs baseline: 11.6824x; 1.0095x over previous
import functools

import jax
import jax.numpy as jnp
from jax.experimental import pallas as pl
from jax.experimental.pallas import tpu as pltpu

_G = 64
_NSLOT = 4


def _pool_kernel(x_hbm, b_ref, w1_ref, b1_ref, w2_ref, b2_ref,
                 acc_out, xbuf, sems, m_ref, d_ref, acc_ref,
                 *, num_segments, block):
    i = pl.program_id(0)
    nb = pl.num_programs(0)
    g = num_segments
    k = _NSLOT

    def copy_in(blk, slot):
        return pltpu.make_async_copy(
            x_hbm.at[pl.ds(blk * block, block), :], xbuf.at[slot],
            sems.at[slot])

    @pl.when(i == 0)
    def _init():
        m_ref[...] = jnp.full(m_ref.shape, -jnp.inf, dtype=jnp.float32)
        d_ref[...] = jnp.zeros(d_ref.shape, dtype=jnp.float32)
        acc_ref[...] = jnp.zeros(acc_ref.shape, dtype=jnp.float32)
        for slot in range(k):
            @pl.when(slot < nb)
            def _():
                copy_in(slot, slot).start()

    slot_i = jax.lax.rem(i, k)
    copy_in(i, slot_i).wait()

    x = xbuf[slot_i]
    xb = x.astype(jnp.bfloat16)
    seg = b_ref[0]
    bsz = block

    h = jnp.tanh(
        jnp.dot(xb, w1_ref[...], preferred_element_type=jnp.float32)
        + b1_ref[...])
    s_t = jax.lax.dot_general(
        w2_ref[...], h, (((1,), (1,)), ((), ())),
        preferred_element_type=jnp.float32) + b2_ref[...]

    blk_max = jnp.max(s_t)
    m_old = m_ref[...]
    m_new = jnp.maximum(m_old, blk_max)
    scale_old = jnp.exp(m_old - m_new)
    scale_blk = jnp.exp(blk_max - m_new)

    e_t = jnp.exp(s_t - blk_max)
    seg_ids = jax.lax.broadcasted_iota(jnp.int32, (g, bsz), 0)
    p = jnp.where(seg_ids == seg, e_t, 0.0).astype(jnp.bfloat16)

    pd = jnp.dot(p, jnp.ones((bsz, 1), jnp.bfloat16),
                 preferred_element_type=jnp.float32)
    pa = jnp.dot(p, xb, preferred_element_type=jnp.float32)

    m_ref[...] = m_new
    d_ref[...] = d_ref[...] * scale_old + scale_blk * pd
    acc_ref[...] = acc_ref[...] * scale_old + scale_blk * pa

    @pl.when(i + k < nb)
    def _refill():
        copy_in(i + k, slot_i).start()

    @pl.when(i == nb - 1)
    def _finish():
        d_fin = d_ref[...]
        acc_out[...] = acc_ref[...] / jnp.where(d_fin > 0, d_fin, 1.0)


def kernel(x, batch, W1, b1, W2, b2):
    n, d = x.shape
    g = _G
    block = 4000
    assert n % block == 0
    nb = n // block

    batch32 = batch.astype(jnp.int32).reshape(nb, 1, block)
    w1b = W1.astype(jnp.bfloat16)
    b1r = b1.reshape(1, d)
    w2r = W2.reshape(1, d)
    b2r = b2.reshape(1, 1)

    grid_spec = pltpu.PrefetchScalarGridSpec(
        num_scalar_prefetch=0,
        grid=(nb,),
        in_specs=[
            pl.BlockSpec(memory_space=pltpu.MemorySpace.HBM),
            pl.BlockSpec((1, 1, block), lambda i: (i, 0, 0)),
            pl.BlockSpec((d, d), lambda i: (0, 0)),
            pl.BlockSpec((1, d), lambda i: (0, 0)),
            pl.BlockSpec((1, d), lambda i: (0, 0)),
            pl.BlockSpec((1, 1), lambda i: (0, 0)),
        ],
        out_specs=pl.BlockSpec((g, d), lambda i: (0, 0)),
        scratch_shapes=[
            pltpu.VMEM((_NSLOT, block, d), jnp.float32),
            pltpu.SemaphoreType.DMA((_NSLOT,)),
            pltpu.VMEM((g, 1), jnp.float32),
            pltpu.VMEM((g, 1), jnp.float32),
            pltpu.VMEM((g, d), jnp.float32),
        ],
    )
    return pl.pallas_call(
        functools.partial(_pool_kernel, num_segments=g, block=block),
        grid_spec=grid_spec,
        out_shape=jax.ShapeDtypeStruct((g, d), jnp.float32),
        compiler_params=pltpu.CompilerParams(
            dimension_semantics=("arbitrary",),
        ),
    )(x, batch32, w1b, b1r, w2r, b2r)


# speedup vs baseline: 11.6868x; 1.0004x over previous
import functools

import jax
import jax.numpy as jnp
from jax.experimental import pallas as pl
from jax.experimental.pallas import tpu as pltpu

_G = 64
_NSLOT = 4


def _pool_kernel(x_hbm, b_ref, w1_ref, b1_ref, w2_ref, b2_ref,
                 acc_out, xbuf, sems, m_ref, d_ref, acc_ref,
                 *, num_segments, block):
    i = pl.program_id(0)
    nb = pl.num_programs(0)
    g = num_segments
    k = _NSLOT

    half = block // 2

    def copies_in(blk, slot):
        return (
            pltpu.make_async_copy(
                x_hbm.at[pl.ds(blk * block, half), :],
                xbuf.at[slot, pl.ds(0, half)], sems.at[slot, 0]),
            pltpu.make_async_copy(
                x_hbm.at[pl.ds(blk * block + half, half), :],
                xbuf.at[slot, pl.ds(half, half)], sems.at[slot, 1]),
        )

    def start_in(blk, slot):
        for c in copies_in(blk, slot):
            c.start()

    def wait_in(blk, slot):
        for c in copies_in(blk, slot):
            c.wait()

    @pl.when(i == 0)
    def _init():
        m_ref[...] = jnp.full(m_ref.shape, -jnp.inf, dtype=jnp.float32)
        d_ref[...] = jnp.zeros(d_ref.shape, dtype=jnp.float32)
        acc_ref[...] = jnp.zeros(acc_ref.shape, dtype=jnp.float32)
        for slot in range(k):
            @pl.when(slot < nb)
            def _():
                start_in(slot, slot)

    slot_i = jax.lax.rem(i, k)
    wait_in(i, slot_i)

    x = xbuf[slot_i]
    xb = x.astype(jnp.bfloat16)
    seg = b_ref[0]
    bsz = block

    h = jnp.tanh(
        jnp.dot(xb, w1_ref[...], preferred_element_type=jnp.float32)
        + b1_ref[...])
    s_t = jax.lax.dot_general(
        w2_ref[...], h, (((1,), (1,)), ((), ())),
        preferred_element_type=jnp.float32) + b2_ref[...]

    blk_max = jnp.max(s_t)
    m_old = m_ref[...]
    m_new = jnp.maximum(m_old, blk_max)
    scale_old = jnp.exp(m_old - m_new)
    scale_blk = jnp.exp(blk_max - m_new)

    e_t = jnp.exp(s_t - blk_max)
    seg_ids = jax.lax.broadcasted_iota(jnp.int32, (g, bsz), 0)
    p = jnp.where(seg_ids == seg, e_t, 0.0).astype(jnp.bfloat16)

    pd = jnp.dot(p, jnp.ones((bsz, 1), jnp.bfloat16),
                 preferred_element_type=jnp.float32)
    pa = jnp.dot(p, xb, preferred_element_type=jnp.float32)

    m_ref[...] = m_new
    d_ref[...] = d_ref[...] * scale_old + scale_blk * pd
    acc_ref[...] = acc_ref[...] * scale_old + scale_blk * pa

    @pl.when(i + k < nb)
    def _refill():
        start_in(i + k, slot_i)

    @pl.when(i == nb - 1)
    def _finish():
        d_fin = d_ref[...]
        acc_out[...] = acc_ref[...] / jnp.where(d_fin > 0, d_fin, 1.0)


def kernel(x, batch, W1, b1, W2, b2):
    n, d = x.shape
    g = _G
    block = 4000
    assert n % block == 0
    nb = n // block

    batch32 = batch.astype(jnp.int32).reshape(nb, 1, block)
    w1b = W1.astype(jnp.bfloat16)
    b1r = b1.reshape(1, d)
    w2r = W2.reshape(1, d)
    b2r = b2.reshape(1, 1)

    grid_spec = pltpu.PrefetchScalarGridSpec(
        num_scalar_prefetch=0,
        grid=(nb,),
        in_specs=[
            pl.BlockSpec(memory_space=pltpu.MemorySpace.HBM),
            pl.BlockSpec((1, 1, block), lambda i: (i, 0, 0)),
            pl.BlockSpec((d, d), lambda i: (0, 0)),
            pl.BlockSpec((1, d), lambda i: (0, 0)),
            pl.BlockSpec((1, d), lambda i: (0, 0)),
            pl.BlockSpec((1, 1), lambda i: (0, 0)),
        ],
        out_specs=pl.BlockSpec((g, d), lambda i: (0, 0)),
        scratch_shapes=[
            pltpu.VMEM((_NSLOT, block, d), jnp.float32),
            pltpu.SemaphoreType.DMA((_NSLOT, 2)),
            pltpu.VMEM((g, 1), jnp.float32),
            pltpu.VMEM((g, 1), jnp.float32),
            pltpu.VMEM((g, d), jnp.float32),
        ],
    )
    return pl.pallas_call(
        functools.partial(_pool_kernel, num_segments=g, block=block),
        grid_spec=grid_spec,
        out_shape=jax.ShapeDtypeStruct((g, d), jnp.float32),
        compiler_params=pltpu.CompilerParams(
            dimension_semantics=("arbitrary",),
        ),
    )(x, batch32, w1b, b1r, w2r, b2r)


# speedup vs baseline: 11.6973x; 1.0009x over previous
import functools

import jax
import jax.numpy as jnp
from jax.experimental import pallas as pl
from jax.experimental.pallas import tpu as pltpu

_G = 64
_NSLOT = 4


def _pool_kernel(x_hbm, b_ref, w1_ref, b1_ref, w2_ref, b2_ref,
                 acc_out, xbuf, sems, m_ref, d_ref, acc_ref,
                 *, num_segments, block):
    i = pl.program_id(0)
    nb = pl.num_programs(0)
    g = num_segments
    k = _NSLOT

    def copy_in(blk, slot):
        return pltpu.make_async_copy(
            x_hbm.at[pl.ds(blk * block, block), :], xbuf.at[slot],
            sems.at[slot])

    @pl.when(i == 0)
    def _init():
        m_ref[...] = jnp.full(m_ref.shape, -jnp.inf, dtype=jnp.float32)
        d_ref[...] = jnp.zeros(d_ref.shape, dtype=jnp.float32)
        acc_ref[...] = jnp.zeros(acc_ref.shape, dtype=jnp.float32)
        for slot in range(k):
            @pl.when(slot < nb)
            def _():
                copy_in(slot, slot).start()

    slot_i = jax.lax.rem(i, k)
    copy_in(i, slot_i).wait()

    x = xbuf[slot_i]
    xb = x.astype(jnp.bfloat16)
    seg = b_ref[0]
    bsz = block

    h = jnp.tanh(
        jnp.dot(xb, w1_ref[...], preferred_element_type=jnp.float32)
        + b1_ref[...])
    s_t = jax.lax.dot_general(
        w2_ref[...], h, (((1,), (1,)), ((), ())),
        preferred_element_type=jnp.float32) + b2_ref[...]

    blk_max = jnp.max(s_t)
    m_old = m_ref[...]
    m_new = jnp.maximum(m_old, blk_max)
    scale_old = jnp.exp(m_old - m_new)
    scale_blk = jnp.exp(blk_max - m_new)

    e_t = jnp.exp(s_t - blk_max)
    seg_ids = jax.lax.broadcasted_iota(jnp.int32, (g, bsz), 0)
    p = jnp.where(seg_ids == seg, e_t, 0.0).astype(jnp.bfloat16)

    pd = jnp.dot(p, jnp.ones((bsz, 1), jnp.bfloat16),
                 preferred_element_type=jnp.float32)
    pa = jnp.dot(p, xb, preferred_element_type=jnp.float32)

    m_ref[...] = m_new
    d_ref[...] = d_ref[...] * scale_old + scale_blk * pd
    acc_ref[...] = acc_ref[...] * scale_old + scale_blk * pa

    @pl.when(i + k < nb)
    def _refill():
        copy_in(i + k, slot_i).start()

    @pl.when(i == nb - 1)
    def _finish():
        d_fin = d_ref[...]
        acc_out[...] = acc_ref[...] / jnp.where(d_fin > 0, d_fin, 1.0)


def kernel(x, batch, W1, b1, W2, b2):
    n, d = x.shape
    g = _G
    block = 4000
    assert n % block == 0
    nb = n // block

    batch32 = batch.astype(jnp.int32).reshape(nb, 1, block)
    w1b = W1.astype(jnp.bfloat16)
    b1r = b1.reshape(1, d)
    w2r = W2.reshape(1, d)
    b2r = b2.reshape(1, 1)

    grid_spec = pltpu.PrefetchScalarGridSpec(
        num_scalar_prefetch=0,
        grid=(nb,),
        in_specs=[
            pl.BlockSpec(memory_space=pltpu.MemorySpace.HBM),
            pl.BlockSpec((1, 1, block), lambda i: (i, 0, 0)),
            pl.BlockSpec((d, d), lambda i: (0, 0)),
            pl.BlockSpec((1, d), lambda i: (0, 0)),
            pl.BlockSpec((1, d), lambda i: (0, 0)),
            pl.BlockSpec((1, 1), lambda i: (0, 0)),
        ],
        out_specs=pl.BlockSpec((g, d), lambda i: (0, 0)),
        scratch_shapes=[
            pltpu.VMEM((_NSLOT, block, d), jnp.float32),
            pltpu.SemaphoreType.DMA((_NSLOT,)),
            pltpu.VMEM((g, 1), jnp.float32),
            pltpu.VMEM((g, 1), jnp.float32),
            pltpu.VMEM((g, d), jnp.float32),
        ],
    )
    return pl.pallas_call(
        functools.partial(_pool_kernel, num_segments=g, block=block),
        grid_spec=grid_spec,
        out_shape=jax.ShapeDtypeStruct((g, d), jnp.float32),
        compiler_params=pltpu.CompilerParams(
            dimension_semantics=("arbitrary",),
        ),
    )(x, batch32, w1b, b1r, w2r, b2r)
